# baseline (speedup 1.0000x reference)
import functools

import jax
import jax.numpy as jnp
from jax import lax
from jax.experimental import pallas as pl
from jax.experimental.pallas import tpu as pltpu
from jax.experimental.pallas import tpu_sc as plsc

IN_DIM = 1000000
HID_C = 32

GW = 128
G = 8
W = G * GW

LB = 8
VC = 32768


def _prep_body(src_ref, tgt_ref, idx_ref, t_ref):
    v = src_ref[...]
    s = v // 20 - 1
    s = jnp.where(s < 0, s + IN_DIM, s)
    rem = s % VC
    s = (s - rem) + 4 * (rem % (VC // 4)) + rem // (VC // 4)
    for l in range(LB):
        for q in range(32):
            idx_ref[32 * l + q : 32 * l + q + 1, :] = (
                s[l : l + 1, 128 * q : 128 * (q + 1)]
            )
    t_ref[...] = tgt_ref[...] // 20 - 1


def _tab_body(tab_ref, w_ref):
    x = tab_ref[...]
    z = jnp.concatenate(
        [x[:, (VC // 4) * u : (VC // 4) * (u + 1)] for u in range(4)], axis=0
    )
    w_ref[...] = jnp.transpose(z)


OLB = 10


def _out_body(rows_ref, emb_ref):
    for p in range(OLB):
        x = rows_ref[1024 * p : 1024 * (p + 1), :]
        for q in range(4):
            xt = jnp.transpose(x[256 * q : 256 * (q + 1), :])
            for u in range(4):
                emb_ref[
                    p, :, 1024 * q + 256 * u : 1024 * q + 256 * (u + 1)
                ] = xt[32 * u : 32 * (u + 1), :]


def _make_gather(n_total: int, v_rows: int):
    assert n_total % W == 0
    mesh = plsc.VectorSubcoreMesh(core_axis_name="c", subcore_axis_name="s")

    @functools.partial(
        pl.kernel,
        mesh=mesh,
        out_type=jax.ShapeDtypeStruct((n_total, HID_C), jnp.float32),
        compiler_params=pltpu.CompilerParams(
            needs_layout_passes=False, use_tc_tiling_on_sc=False
        ),
        scratch_types=[
            pltpu.VMEM((G, GW), jnp.int32),
            pltpu.SemaphoreType.DMA,
        ],
    )
    def gather_kernel(idx_hbm, table_hbm, out_hbm, idx2, sem):
        def body(idx_vmem, out_vmem):
            for w in range(G):
                for k in range(GW // 16):
                    c = lax.iota(jnp.int32, 16) + 16 * k
                    rowv = 2 * (c % 4) + (w // 4)
                    colv = 32 * (w % 4) + c // 4
                    idx2[w, pl.ds(16 * k, 16)] = plsc.load_gather(
                        idx_vmem, [rowv, colv]
                    )
            copies = [
                pltpu.async_copy(
                    table_hbm.at[idx2.at[g]],
                    out_vmem.at[pl.ds(g * GW, GW)],
                    sem,
                )
                for g in range(G)
            ]
            for cp in copies:
                cp.wait()

        pltpu.emit_pipeline(
            body,
            grid=(n_total // W,),
            in_specs=[pl.BlockSpec((G, GW), index_map=lambda i: (i, 0))],
            out_specs=[pl.BlockSpec((W, HID_C), index_map=lambda i: (i, 0))],
            core_axis_name=("c", "s"),
            dimension_semantics=(pltpu.PARALLEL,),
        )(idx_hbm, out_hbm)

    return gather_kernel


@jax.jit
def kernel(source, target, table):
    b, seq = source.shape
    n_total = b * seq
    v_dim = table.shape[0]
    nvb = (v_dim + VC - 1) // VC

    src_t = source.T
    tgt_t = target.T

    idx_flat, t_t = pl.pallas_call(
        _prep_body,
        grid=(seq // LB,),
        in_specs=[
            pl.BlockSpec((LB, b), lambda i: (i, 0)),
            pl.BlockSpec((LB, b), lambda i: (i, 0)),
        ],
        out_specs=[
            pl.BlockSpec((LB * b // 128, 128), lambda i: (i, 0)),
            pl.BlockSpec((LB, b), lambda i: (i, 0)),
        ],
        out_shape=[
            jax.ShapeDtypeStruct((n_total // 128, 128), jnp.int32),
            jax.ShapeDtypeStruct((seq, b), target.dtype),
        ],
        compiler_params=pltpu.CompilerParams(
            dimension_semantics=("parallel",)
        ),
    )(src_t, tgt_t)

    w_tab = pl.pallas_call(
        _tab_body,
        grid=(nvb,),
        in_specs=[pl.BlockSpec((HID_C, VC), lambda i: (0, i))],
        out_specs=pl.BlockSpec((VC // 4, 128), lambda i: (i, 0)),
        out_shape=jax.ShapeDtypeStruct((nvb * VC // 4, 128), jnp.float32),
        compiler_params=pltpu.CompilerParams(
            dimension_semantics=("parallel",)
        ),
    )(table.T)

    rows = _make_gather(n_total, nvb * VC)(
        idx_flat, w_tab.reshape(nvb * VC, HID_C)
    )

    emb_t = pl.pallas_call(
        _out_body,
        grid=(seq // OLB,),
        in_specs=[pl.BlockSpec((OLB * b * HID_C // 128, 128), lambda i: (i, 0))],
        out_specs=pl.BlockSpec((OLB, HID_C, b), lambda i: (i, 0, 0)),
        out_shape=jax.ShapeDtypeStruct((seq, HID_C, b), jnp.float32),
        compiler_params=pltpu.CompilerParams(
            dimension_semantics=("parallel",)
        ),
    )(rows.reshape(n_total * HID_C // 128, 128))

    return (jnp.transpose(emb_t, (2, 0, 1)), t_t.T)
